# baseline (speedup 1.0000x reference)
import functools

import jax
import jax.numpy as jnp
from jax import lax
from jax.experimental import pallas as pl
from jax.experimental.pallas import tpu as pltpu
from jax.experimental.pallas import tpu_sc as plsc

_VOCAB = 100000
_EMB_DIM = 64
_DEFAULT_IDX = 0
_LANES = 16
_BATCH = 4096
_HIST = 50
_NC = 2
_NS = 16
_BW = _BATCH // (_NC * _NS)


def _sc_gather(table2, idx_t):
    mesh = plsc.VectorSubcoreMesh(core_axis_name="c", subcore_axis_name="s")

    nbuf = 6

    @functools.partial(
        pl.kernel,
        out_type=jax.ShapeDtypeStruct((_HIST, _BATCH, 2 * _EMB_DIM),
                                      jnp.float32),
        mesh=mesh,
        scratch_types=[
            pltpu.VMEM((_HIST, _BW), jnp.int32),
            pltpu.VMEM((_BW, 2 * _EMB_DIM), jnp.float32),
            pltpu.VMEM((_BW, 2 * _EMB_DIM), jnp.float32),
            pltpu.VMEM((_BW, 2 * _EMB_DIM), jnp.float32),
            pltpu.VMEM((_BW, 2 * _EMB_DIM), jnp.float32),
            pltpu.VMEM((_BW, 2 * _EMB_DIM), jnp.float32),
            pltpu.VMEM((_BW, 2 * _EMB_DIM), jnp.float32),
            pltpu.SemaphoreType.DMA,
            pltpu.SemaphoreType.DMA,
        ],
        compiler_params=pltpu.CompilerParams(use_tc_tiling_on_sc=False),
    )
    def k(table_hbm, idx_hbm, x_hbm, idx_v, s0, s1, s2, s3, s4, s5, sem, isem):
        wid = lax.axis_index("c") * _NS + lax.axis_index("s")
        b0 = wid * _BW
        bufs = [s0, s1, s2, s3, s4, s5]

        ld = [pltpu.async_copy(idx_hbm.at[pl.ds(h * _BATCH + b0, _BW)],
                               idx_v.at[h], isem)
              for h in range(_HIST)]
        for c in ld:
            c.wait()

        def clamp_body(h, carry):
            for l in range(_BW // _LANES):
                sl = pl.ds(l * _LANES, _LANES)
                v = idx_v[h, sl]
                ok = (v >= 0) & (v < _VOCAB)
                idx_v[h, sl] = jnp.where(ok, v, _DEFAULT_IDX)
            return carry

        lax.fori_loop(0, _HIST, clamp_body, 0)

        for b in range(nbuf):
            pltpu.async_copy(table_hbm.at[idx_v.at[b]], bufs[b], sem)

        def pipe_body(j, carry):
            for b in range(nbuf):
                h = nbuf * j + b
                pltpu.make_async_copy(
                    table_hbm.at[idx_v.at[0]], bufs[b], sem).wait()
                pltpu.sync_copy(bufs[b], x_hbm.at[h, pl.ds(b0, _BW)])

                @pl.when(h + nbuf < _HIST)
                def _():
                    pltpu.async_copy(
                        table_hbm.at[idx_v.at[h + nbuf]], bufs[b], sem)
            return carry

        lax.fori_loop(0, _HIST // nbuf, pipe_body, 0)
        for b in range(_HIST % nbuf):
            h = _HIST - _HIST % nbuf + b
            pltpu.make_async_copy(
                table_hbm.at[idx_v.at[0]], bufs[b], sem).wait()
            pltpu.sync_copy(bufs[b], x_hbm.at[h, pl.ds(b0, _BW)])

    return k(table2, idx_t)


def _tc_transpose(x):
    def body(x_ref, y_ref):
        xb = x_ref[0]
        y_ref[0] = jnp.transpose(xb[:, :_EMB_DIM], (1, 0))

    return pl.pallas_call(
        body,
        grid=(_HIST,),
        in_specs=[pl.BlockSpec((1, _BATCH, 2 * _EMB_DIM),
                               lambda i: (i, 0, 0))],
        out_specs=pl.BlockSpec((1, _EMB_DIM, _BATCH), lambda i: (i, 0, 0)),
        out_shape=jax.ShapeDtypeStruct((_HIST, _EMB_DIM, _BATCH),
                                       jnp.float32),
    )(x)


def kernel(inputs, embeddings):
    table2 = jnp.pad(
        embeddings, ((0, 0), (0, 2 * _EMB_DIM - embeddings.shape[1])))
    idx_t = inputs.T.reshape(-1)
    x = _sc_gather(table2, idx_t)
    y = _tc_transpose(x)
    return jnp.transpose(y, (2, 0, 1))


# speedup vs baseline: 1.0132x; 1.0132x over previous
import functools

import jax
import jax.numpy as jnp
from jax import lax
from jax.experimental import pallas as pl
from jax.experimental.pallas import tpu as pltpu
from jax.experimental.pallas import tpu_sc as plsc

_VOCAB = 100000
_EMB_DIM = 64
_DEFAULT_IDX = 0
_LANES = 16
_BATCH = 4096
_HIST = 50
_NC = 2
_NS = 16
_BW = _BATCH // (_NC * _NS)


def _sc_gather(table2, idx_t):
    mesh = plsc.VectorSubcoreMesh(core_axis_name="c", subcore_axis_name="s")

    nbuf = 6

    @functools.partial(
        pl.kernel,
        out_type=jax.ShapeDtypeStruct((_HIST, _BATCH, 2 * _EMB_DIM),
                                      jnp.float32),
        mesh=mesh,
        scratch_types=[
            pltpu.VMEM((_HIST, _BW), jnp.int32),
            pltpu.VMEM((_BW, 2 * _EMB_DIM), jnp.float32),
            pltpu.VMEM((_BW, 2 * _EMB_DIM), jnp.float32),
            pltpu.VMEM((_BW, 2 * _EMB_DIM), jnp.float32),
            pltpu.VMEM((_BW, 2 * _EMB_DIM), jnp.float32),
            pltpu.VMEM((_BW, 2 * _EMB_DIM), jnp.float32),
            pltpu.VMEM((_BW, 2 * _EMB_DIM), jnp.float32),
            pltpu.SemaphoreType.DMA,
            pltpu.SemaphoreType.DMA,
        ],
        compiler_params=pltpu.CompilerParams(use_tc_tiling_on_sc=False),
    )
    def k(table_hbm, idx_hbm, x_hbm, idx_v, s0, s1, s2, s3, s4, s5, sem, isem):
        wid = lax.axis_index("c") * _NS + lax.axis_index("s")
        b0 = wid * _BW
        bufs = [s0, s1, s2, s3, s4, s5]

        ld = [pltpu.async_copy(idx_hbm.at[pl.ds(h * _BATCH + b0, _BW)],
                               idx_v.at[h], isem)
              for h in range(_HIST)]
        for c in ld:
            c.wait()

        def clamp_body(h, carry):
            for l in range(_BW // _LANES):
                sl = pl.ds(l * _LANES, _LANES)
                v = idx_v[h, sl]
                ok = (v >= 0) & (v < _VOCAB)
                idx_v[h, sl] = jnp.where(ok, v, _DEFAULT_IDX)
            return carry

        lax.fori_loop(0, _HIST, clamp_body, 0)

        for b in range(nbuf):
            pltpu.async_copy(table_hbm.at[idx_v.at[b]], bufs[b], sem)

        def pipe_body(j, carry):
            for b in range(nbuf):
                h = nbuf * j + b
                pltpu.make_async_copy(
                    table_hbm.at[idx_v.at[0]], bufs[b], sem).wait()
                pltpu.sync_copy(bufs[b], x_hbm.at[h, pl.ds(b0, _BW)])

                @pl.when(h + nbuf < _HIST)
                def _():
                    pltpu.async_copy(
                        table_hbm.at[idx_v.at[h + nbuf]], bufs[b], sem)
            return carry

        lax.fori_loop(0, _HIST // nbuf, pipe_body, 0)
        for b in range(_HIST % nbuf):
            h = _HIST - _HIST % nbuf + b
            pltpu.make_async_copy(
                table_hbm.at[idx_v.at[0]], bufs[b], sem).wait()
            pltpu.sync_copy(bufs[b], x_hbm.at[h, pl.ds(b0, _BW)])

    return k(table2, idx_t)


def _tc_transpose(x):
    def body(x_ref, y_ref):
        xb = x_ref[0]
        eye = jnp.eye(_EMB_DIM, dtype=jnp.float32)
        y_ref[0] = jax.lax.dot_general(
            eye, xb[:, :_EMB_DIM], (((1,), (1,)), ((), ())),
            preferred_element_type=jnp.float32)

    return pl.pallas_call(
        body,
        grid=(_HIST,),
        in_specs=[pl.BlockSpec((1, _BATCH, 2 * _EMB_DIM),
                               lambda i: (i, 0, 0))],
        out_specs=pl.BlockSpec((1, _EMB_DIM, _BATCH), lambda i: (i, 0, 0)),
        out_shape=jax.ShapeDtypeStruct((_HIST, _EMB_DIM, _BATCH),
                                       jnp.float32),
    )(x)


def kernel(inputs, embeddings):
    table2 = jnp.pad(
        embeddings, ((0, 0), (0, 2 * _EMB_DIM - embeddings.shape[1])))
    idx_t = inputs.T.reshape(-1)
    x = _sc_gather(table2, idx_t)
    y = _tc_transpose(x)
    return jnp.transpose(y, (2, 0, 1))
